# baseline (speedup 1.0000x reference)
import functools

import jax
import jax.numpy as jnp
from jax import lax
from jax.experimental import pallas as pl
from jax.experimental.pallas import tpu as pltpu
from jax.experimental.pallas import tpu_sc as plsc

B, N = 128, 32768
NSC = 6144
NC, NS = 2, 16
NW = NC * NS
TR = B // 8
HALF = NSC // 2
CW = 1024
NCHUNK = HALF // CW
L = 16

BW = 512
OFF = NSC // BW
NB = (N - NSC) // BW

_COMBOS = [(v1, v2, v3)
           for v3 in (False, True)
           for v2 in (False, True)
           for v1 in (False, True)]


def _node(a, b, mulm, sign, is_div):
    if is_div:
        return a / (b + 1e-06)
    return jnp.where(mulm, a * b, a + sign * b)


def _sc_body(g1, g2, g3, g4, aux, out,
             a10, a20, a30, a40, a11, a21, a31, a41,
             o0, o1, auxv,
             sin0, sin1, sout0, sout1):
    ins = ((a10, a20, a30, a40), (a11, a21, a31, a41))
    outs = (o0, o1)
    sem_in = (sin0, sin1)
    sem_out = (sout0, sout1)
    srcs = (g1, g2, g3, g4)

    wid = lax.axis_index("s") * NC + lax.axis_index("c")
    r0 = lax.rem(wid, TR) * 8
    c0 = lax.div(wid, TR) * HALF

    pltpu.sync_copy(aux, auxv)
    sel1 = auxv[0, pl.ds(0, L)]
    sel2 = auxv[1, pl.ds(0, L)]
    sel3 = auxv[2, pl.ds(0, L)]
    m1, m2, m3 = (s == 2 for s in (sel1, sel2, sel3))
    d1, d2, d3 = (s[0] == 3 for s in (sel1, sel2, sel3))
    one = jnp.float32(1.0)
    sg1, sg2, sg3 = (jnp.where(s == 1, -one, one) for s in (sel1, sel2, sel3))

    def start_in(b, chunk):
        c = c0 + chunk * CW
        for g, dst in zip(srcs, ins[b]):
            pltpu.async_copy(g.at[pl.ds(r0, 8), pl.ds(c, CW)], dst, sem_in[b])

    def wait_in(b, chunk):
        c = c0 + chunk * CW
        for g, dst in zip(srcs, ins[b]):
            pltpu.make_async_copy(
                g.at[pl.ds(r0, 8), pl.ds(c, CW)], dst, sem_in[b]).wait()

    start_in(0, 0)

    def compute(b, v1, v2, v3):
        g1b, g2b, g3b, g4b = ins[b]
        ob = outs[b]

        def body(i, carry):
            sl = pl.ds(i * L, L)
            for r in range(8):
                x1 = g1b[r, sl]
                x2 = g2b[r, sl]
                x3 = g3b[r, sl]
                x4 = g4b[r, sl]
                t1 = _node(x1, x2, m1, sg1, v1)
                t2 = _node(x3, x4, m2, sg2, v2)
                ob[r, sl] = _node(t1, t2, m3, sg3, v3)
            return carry
        lax.fori_loop(0, CW // L, body, 0)

    for chunk in range(NCHUNK):
        b = chunk % 2
        wait_in(b, chunk)
        if chunk + 1 < NCHUNK:
            start_in(1 - b, chunk + 1)
        if chunk >= 2:
            c_prev = c0 + (chunk - 2) * CW
            pltpu.make_async_copy(
                outs[b], out.at[pl.ds(r0, 8), pl.ds(c_prev, CW)],
                sem_out[b]).wait()
        for v1, v2, v3 in _COMBOS:
            cond = ((d1 == v1) & (d2 == v2) & (d3 == v3))
            pl.when(cond)(functools.partial(compute, b, v1, v2, v3))
        c = c0 + chunk * CW
        pltpu.async_copy(outs[b], out.at[pl.ds(r0, 8), pl.ds(c, CW)],
                         sem_out[b])

    for chunk in (NCHUNK - 2, NCHUNK - 1):
        b = chunk % 2
        c = c0 + chunk * CW
        pltpu.make_async_copy(
            outs[b], out.at[pl.ds(r0, 8), pl.ds(c, CW)], sem_out[b]).wait()


_sc_fused = functools.partial(
    pl.kernel,
    out_type=jax.ShapeDtypeStruct((B, NSC), jnp.float32),
    mesh=plsc.VectorSubcoreMesh(core_axis_name="c", subcore_axis_name="s"),
    scratch_types=(
        [pltpu.VMEM((8, CW), jnp.float32) for _ in range(10)]
        + [pltpu.VMEM((8, 128), jnp.int32)]
        + [pltpu.SemaphoreType.DMA for _ in range(4)]
    ),
)(_sc_body)


def _tc_body(v1, v2, v3, idx_ref, g1_ref, g2_ref, g3_ref, g4_ref, out_ref):
    s1 = idx_ref[0]
    s2 = idx_ref[1]
    s3 = idx_ref[2]
    one = jnp.float32(1.0)
    m1, m2, m3 = (s == 2 for s in (s1, s2, s3))
    sg1, sg2, sg3 = (jnp.where(s == 1, -one, one) for s in (s1, s2, s3))

    x1 = g1_ref[...]
    x2 = g2_ref[...]
    x3 = g3_ref[...]
    x4 = g4_ref[...]
    t1 = _node(x1, x2, m1, sg1, v1)
    t2 = _node(x3, x4, m2, sg2, v2)
    out_ref[...] = _node(t1, t2, m3, sg3, v3)


def _make_tc_variant(combo):
    v1, v2, v3 = combo
    return pl.pallas_call(
        functools.partial(_tc_body, v1, v2, v3),
        grid=(NB,),
        in_specs=[
            pl.BlockSpec(memory_space=pltpu.SMEM),
        ] + [
            pl.BlockSpec((B, BW), lambda i: (0, i + OFF)) for _ in range(4)
        ],
        out_specs=pl.BlockSpec((B, BW), lambda i: (0, i + OFF)),
        out_shape=jax.ShapeDtypeStruct((B, N), jnp.float32),
        compiler_params=pltpu.CompilerParams(
            dimension_semantics=("parallel",)),
    )


_TC_VARIANTS = [_make_tc_variant(c) for c in _COMBOS]


def kernel(g1, g2, g3, g4, w1, w2, w3):
    root = jax.random.key(42)
    keys = jax.random.split(root, 3)
    ws = jnp.stack([w1, w2, w3])
    idx = jax.vmap(jax.random.categorical)(keys, ws)
    log_probs = jnp.take_along_axis(
        jax.nn.log_softmax(ws, axis=-1), idx[:, None], axis=-1)[:, 0]

    aux = jnp.zeros((8, 128), jnp.int32).at[0:3, :].set(idx[:, None])
    sc_part = _sc_fused(g1, g2, g3, g4, aux)

    isdiv = (idx == 3).astype(jnp.int32)
    combo = isdiv[0] + 2 * isdiv[1] + 4 * isdiv[2]
    tc_full = lax.switch(
        combo,
        [functools.partial(v, idx) for v in _TC_VARIANTS],
        g1, g2, g3, g4)

    out = lax.dynamic_update_slice(tc_full, sc_part, (0, 0))
    return (out, log_probs, idx)


# speedup vs baseline: 1.4141x; 1.4141x over previous
import functools

import jax
import jax.numpy as jnp
from jax import lax
from jax.experimental import pallas as pl
from jax.experimental.pallas import tpu as pltpu
from jax.experimental.pallas import tpu_sc as plsc

B, N = 128, 32768
NSC = 6144
NC, NS = 2, 16
NW = NC * NS
TR = B // 8
HALF = NSC // 2
CW = 1024
NCHUNK = HALF // CW
L = 16

BW = 2048
OFF = NSC // BW
NB = (N - NSC) // BW

_COMBOS = [(v1, v2, v3)
           for v3 in (False, True)
           for v2 in (False, True)
           for v1 in (False, True)]


def _node(a, b, mulm, sign, is_div):
    if is_div:
        return a / (b + 1e-06)
    return jnp.where(mulm, a * b, a + sign * b)


def _sc_body(g1, g2, g3, g4, aux, out,
             a10, a20, a30, a40, a11, a21, a31, a41,
             o0, o1, auxv,
             sin0, sin1, sout0, sout1):
    ins = ((a10, a20, a30, a40), (a11, a21, a31, a41))
    outs = (o0, o1)
    sem_in = (sin0, sin1)
    sem_out = (sout0, sout1)
    srcs = (g1, g2, g3, g4)

    wid = lax.axis_index("s") * NC + lax.axis_index("c")
    r0 = lax.rem(wid, TR) * 8
    c0 = lax.div(wid, TR) * HALF

    pltpu.sync_copy(aux, auxv)
    sel1 = auxv[0, pl.ds(0, L)]
    sel2 = auxv[1, pl.ds(0, L)]
    sel3 = auxv[2, pl.ds(0, L)]
    m1, m2, m3 = (s == 2 for s in (sel1, sel2, sel3))
    d1, d2, d3 = (s[0] == 3 for s in (sel1, sel2, sel3))
    one = jnp.float32(1.0)
    sg1, sg2, sg3 = (jnp.where(s == 1, -one, one) for s in (sel1, sel2, sel3))

    def start_in(b, chunk):
        c = c0 + chunk * CW
        for g, dst in zip(srcs, ins[b]):
            pltpu.async_copy(g.at[pl.ds(r0, 8), pl.ds(c, CW)], dst, sem_in[b])

    def wait_in(b, chunk):
        c = c0 + chunk * CW
        for g, dst in zip(srcs, ins[b]):
            pltpu.make_async_copy(
                g.at[pl.ds(r0, 8), pl.ds(c, CW)], dst, sem_in[b]).wait()

    start_in(0, 0)

    def compute(b, v1, v2, v3):
        g1b, g2b, g3b, g4b = ins[b]
        ob = outs[b]

        def body(i, carry):
            sl = pl.ds(i * L, L)
            for r in range(8):
                x1 = g1b[r, sl]
                x2 = g2b[r, sl]
                x3 = g3b[r, sl]
                x4 = g4b[r, sl]
                t1 = _node(x1, x2, m1, sg1, v1)
                t2 = _node(x3, x4, m2, sg2, v2)
                ob[r, sl] = _node(t1, t2, m3, sg3, v3)
            return carry
        lax.fori_loop(0, CW // L, body, 0)

    for chunk in range(NCHUNK):
        b = chunk % 2
        wait_in(b, chunk)
        if chunk + 1 < NCHUNK:
            start_in(1 - b, chunk + 1)
        if chunk >= 2:
            c_prev = c0 + (chunk - 2) * CW
            pltpu.make_async_copy(
                outs[b], out.at[pl.ds(r0, 8), pl.ds(c_prev, CW)],
                sem_out[b]).wait()
        for v1, v2, v3 in _COMBOS:
            cond = ((d1 == v1) & (d2 == v2) & (d3 == v3))
            pl.when(cond)(functools.partial(compute, b, v1, v2, v3))
        c = c0 + chunk * CW
        pltpu.async_copy(outs[b], out.at[pl.ds(r0, 8), pl.ds(c, CW)],
                         sem_out[b])

    for chunk in (NCHUNK - 2, NCHUNK - 1):
        b = chunk % 2
        c = c0 + chunk * CW
        pltpu.make_async_copy(
            outs[b], out.at[pl.ds(r0, 8), pl.ds(c, CW)], sem_out[b]).wait()


_sc_fused = functools.partial(
    pl.kernel,
    out_type=jax.ShapeDtypeStruct((B, NSC), jnp.float32),
    mesh=plsc.VectorSubcoreMesh(core_axis_name="c", subcore_axis_name="s"),
    scratch_types=(
        [pltpu.VMEM((8, CW), jnp.float32) for _ in range(10)]
        + [pltpu.VMEM((8, 128), jnp.int32)]
        + [pltpu.SemaphoreType.DMA for _ in range(4)]
    ),
)(_sc_body)


def _tc_body(idx_ref, g1_ref, g2_ref, g3_ref, g4_ref, out_ref):
    s1 = idx_ref[0]
    s2 = idx_ref[1]
    s3 = idx_ref[2]
    one = jnp.float32(1.0)
    m1, m2, m3 = (s == 2 for s in (s1, s2, s3))
    sg1, sg2, sg3 = (jnp.where(s == 1, -one, one) for s in (s1, s2, s3))

    def compute(v1, v2, v3):
        x1 = g1_ref[...]
        x2 = g2_ref[...]
        x3 = g3_ref[...]
        x4 = g4_ref[...]
        t1 = _node(x1, x2, m1, sg1, v1)
        t2 = _node(x3, x4, m2, sg2, v2)
        out_ref[...] = _node(t1, t2, m3, sg3, v3)

    for v1, v2, v3 in _COMBOS:
        cond = (((s1 == 3) == v1) & ((s2 == 3) == v2)
                & ((s3 == 3) == v3))
        pl.when(cond)(functools.partial(compute, v1, v2, v3))


_tc_fused = pl.pallas_call(
    _tc_body,
    grid=(NB,),
    in_specs=[
        pl.BlockSpec(memory_space=pltpu.SMEM),
    ] + [
        pl.BlockSpec((B, BW), lambda i: (0, i + OFF)) for _ in range(4)
    ],
    out_specs=pl.BlockSpec((B, BW), lambda i: (0, i + OFF)),
    out_shape=jax.ShapeDtypeStruct((B, N), jnp.float32),
    compiler_params=pltpu.CompilerParams(
        dimension_semantics=("arbitrary",)),
)


def kernel(g1, g2, g3, g4, w1, w2, w3):
    root = jax.random.key(42)
    keys = jax.random.split(root, 3)
    ws = jnp.stack([w1, w2, w3])
    idx = jax.vmap(jax.random.categorical)(keys, ws)
    log_probs = jnp.take_along_axis(
        jax.nn.log_softmax(ws, axis=-1), idx[:, None], axis=-1)[:, 0]

    aux = jnp.zeros((8, 128), jnp.int32).at[0:3, :].set(idx[:, None])
    sc_part = _sc_fused(g1, g2, g3, g4, aux)

    tc_full = _tc_fused(idx, g1, g2, g3, g4)

    out = lax.dynamic_update_slice(tc_full, sc_part, (0, 0))
    return (out, log_probs, idx)
